# baseline (speedup 1.0000x reference)
import jax
import jax.numpy as jnp
from jax.experimental import pallas as pl
from jax.experimental.pallas import tpu as pltpu

_BM = 40
_NBUF = 16
_CORES = 2


def _gcn_kernel(adj_hbm, x_ref, w_ref, b_ref, o_ref, buf, xw_ref, sems):
    n = x_ref.shape[0]
    rows_per_core = n // _CORES
    nchunks = rows_per_core // _BM
    base = pl.program_id(0) * rows_per_core

    def start(k):
        pltpu.make_async_copy(
            adj_hbm.at[pl.ds(base + k * _BM, _BM), :],
            buf.at[k % _NBUF],
            sems.at[k % _NBUF],
        ).start()

    for k in range(_NBUF - 1):
        start(k)

    xw_ref[...] = jnp.dot(
        x_ref[...], w_ref[...], preferred_element_type=jnp.float32
    )

    for k in range(nchunks):
        if k + _NBUF - 1 < nchunks:
            start(k + _NBUF - 1)
        pltpu.make_async_copy(
            adj_hbm.at[pl.ds(base + k * _BM, _BM), :],
            buf.at[k % _NBUF],
            sems.at[k % _NBUF],
        ).wait()
        acc = jnp.dot(
            buf[k % _NBUF], xw_ref[...], preferred_element_type=jnp.float32
        )
        o_ref[k * _BM:(k + 1) * _BM, :] = acc + b_ref[...]


def kernel(x, adj, w, b):
    n, f = x.shape
    h = w.shape[1]
    b2 = b.reshape(1, h)
    return pl.pallas_call(
        _gcn_kernel,
        grid=(_CORES,),
        in_specs=[
            pl.BlockSpec(memory_space=pl.ANY),
            pl.BlockSpec((n, f), lambda i: (0, 0)),
            pl.BlockSpec((f, h), lambda i: (0, 0)),
            pl.BlockSpec((1, h), lambda i: (0, 0)),
        ],
        out_specs=pl.BlockSpec((n // _CORES, h), lambda i: (i, 0)),
        out_shape=jax.ShapeDtypeStruct((n, h), jnp.float32),
        scratch_shapes=[
            pltpu.VMEM((_NBUF, _BM, n), jnp.float32),
            pltpu.VMEM((n, h), jnp.float32),
            pltpu.SemaphoreType.DMA((_NBUF,)),
        ],
        compiler_params=pltpu.CompilerParams(
            dimension_semantics=("parallel",),
        ),
    )(adj, x, w, b2)


# speedup vs baseline: 1.0594x; 1.0594x over previous
import jax
import jax.numpy as jnp
from jax.experimental import pallas as pl
from jax.experimental.pallas import tpu as pltpu

_BM = 200
_NBUF = 5
_CORES = 2


def _gcn_kernel(adj_hbm, x_ref, w_ref, b_ref, o_ref, buf, xw_ref, sems):
    n = x_ref.shape[0]
    rows_per_core = n // _CORES
    nchunks = rows_per_core // _BM
    base = pl.program_id(0) * rows_per_core

    def start(k):
        pltpu.make_async_copy(
            adj_hbm.at[pl.ds(base + k * _BM, _BM), :],
            buf.at[k % _NBUF],
            sems.at[k % _NBUF],
        ).start()

    for k in range(_NBUF - 1):
        start(k)

    xw_ref[...] = jnp.dot(
        x_ref[...], w_ref[...], preferred_element_type=jnp.float32
    )

    for k in range(nchunks):
        if k + _NBUF - 1 < nchunks:
            start(k + _NBUF - 1)
        pltpu.make_async_copy(
            adj_hbm.at[pl.ds(base + k * _BM, _BM), :],
            buf.at[k % _NBUF],
            sems.at[k % _NBUF],
        ).wait()
        acc = jnp.dot(
            buf[k % _NBUF], xw_ref[...], preferred_element_type=jnp.float32
        )
        o_ref[k * _BM:(k + 1) * _BM, :] = acc + b_ref[...]


def kernel(x, adj, w, b):
    n, f = x.shape
    h = w.shape[1]
    b2 = b.reshape(1, h)
    return pl.pallas_call(
        _gcn_kernel,
        grid=(_CORES,),
        in_specs=[
            pl.BlockSpec(memory_space=pl.ANY),
            pl.BlockSpec((n, f), lambda i: (0, 0)),
            pl.BlockSpec((f, h), lambda i: (0, 0)),
            pl.BlockSpec((1, h), lambda i: (0, 0)),
        ],
        out_specs=pl.BlockSpec((n // _CORES, h), lambda i: (i, 0)),
        out_shape=jax.ShapeDtypeStruct((n, h), jnp.float32),
        scratch_shapes=[
            pltpu.VMEM((_NBUF, _BM, n), jnp.float32),
            pltpu.VMEM((n, h), jnp.float32),
            pltpu.SemaphoreType.DMA((_NBUF,)),
        ],
        compiler_params=pltpu.CompilerParams(
            dimension_semantics=("parallel",),
        ),
    )(adj, x, w, b2)


# speedup vs baseline: 1.0673x; 1.0075x over previous
import jax
import jax.numpy as jnp
from jax.experimental import pallas as pl
from jax.experimental.pallas import tpu as pltpu

_BM = 200
_NBUF = 4
_CORES = 2


def _gcn_kernel(adj_hbm, x_ref, w_ref, b_ref, o_ref, buf, xw_ref, sems):
    n = x_ref.shape[0]
    rows_per_core = n // _CORES
    nchunks = rows_per_core // _BM
    base = pl.program_id(0) * rows_per_core

    def start(k):
        pltpu.make_async_copy(
            adj_hbm.at[pl.ds(base + k * _BM, _BM), :],
            buf.at[k % _NBUF],
            sems.at[k % _NBUF],
        ).start()

    for k in range(_NBUF - 1):
        start(k)

    xw_ref[...] = jnp.dot(
        x_ref[...], w_ref[...], preferred_element_type=jnp.float32
    )

    for k in range(nchunks):
        if k + _NBUF - 1 < nchunks:
            start(k + _NBUF - 1)
        pltpu.make_async_copy(
            adj_hbm.at[pl.ds(base + k * _BM, _BM), :],
            buf.at[k % _NBUF],
            sems.at[k % _NBUF],
        ).wait()
        acc = jnp.dot(
            buf[k % _NBUF], xw_ref[...], preferred_element_type=jnp.float32
        )
        o_ref[k * _BM:(k + 1) * _BM, :] = acc + b_ref[...]


def kernel(x, adj, w, b):
    n, f = x.shape
    h = w.shape[1]
    b2 = b.reshape(1, h)
    return pl.pallas_call(
        _gcn_kernel,
        grid=(_CORES,),
        in_specs=[
            pl.BlockSpec(memory_space=pl.ANY),
            pl.BlockSpec((n, f), lambda i: (0, 0)),
            pl.BlockSpec((f, h), lambda i: (0, 0)),
            pl.BlockSpec((1, h), lambda i: (0, 0)),
        ],
        out_specs=pl.BlockSpec((n // _CORES, h), lambda i: (i, 0)),
        out_shape=jax.ShapeDtypeStruct((n, h), jnp.float32),
        scratch_shapes=[
            pltpu.VMEM((_NBUF, _BM, n), jnp.float32),
            pltpu.VMEM((n, h), jnp.float32),
            pltpu.SemaphoreType.DMA((_NBUF,)),
        ],
        compiler_params=pltpu.CompilerParams(
            dimension_semantics=("parallel",),
        ),
    )(adj, x, w, b2)


# speedup vs baseline: 1.3490x; 1.2640x over previous
import jax
import jax.numpy as jnp
from jax.experimental import pallas as pl
from jax.experimental.pallas import tpu as pltpu


def _xw_kernel(x_ref, w_ref, o_ref):
    o_ref[...] = jnp.dot(
        x_ref[...], w_ref[...], preferred_element_type=jnp.float32
    )


def _spmm_kernel(adj_ref, xw_ref, b_ref, o_ref):
    acc = jax.lax.dot_general(
        adj_ref[...],
        xw_ref[...],
        (((1,), (0,)), ((), ())),
        precision=jax.lax.Precision.DEFAULT,
        preferred_element_type=jnp.float32,
    )
    o_ref[...] = acc + b_ref[...]


def kernel(x, adj, w, b):
    n, f = x.shape
    h = w.shape[1]
    xw = pl.pallas_call(
        _xw_kernel,
        out_shape=jax.ShapeDtypeStruct((n, h), jnp.float32),
    )(x, w)

    bm = 400
    b2 = b.reshape(1, h)
    out = pl.pallas_call(
        _spmm_kernel,
        grid=(pl.cdiv(n, bm),),
        in_specs=[
            pl.BlockSpec((bm, n), lambda i: (i, 0)),
            pl.BlockSpec((n, h), lambda i: (0, 0)),
            pl.BlockSpec((1, h), lambda i: (0, 0)),
        ],
        out_specs=pl.BlockSpec((bm, h), lambda i: (i, 0)),
        out_shape=jax.ShapeDtypeStruct((n, h), jnp.float32),
        compiler_params=pltpu.CompilerParams(
            dimension_semantics=("parallel",),
        ),
    )(adj, xw, b2)
    return out


# speedup vs baseline: 1.3524x; 1.0025x over previous
import jax
import jax.numpy as jnp
from jax.experimental import pallas as pl
from jax.experimental.pallas import tpu as pltpu


def _xw_kernel(x_ref, w_ref, o_ref):
    o_ref[...] = jnp.dot(
        x_ref[...], w_ref[...], preferred_element_type=jnp.float32
    )


def _spmm_kernel(adj_ref, xw_ref, b_ref, o_ref):
    acc = jax.lax.dot_general(
        adj_ref[...],
        xw_ref[...],
        (((1,), (0,)), ((), ())),
        precision=jax.lax.Precision.DEFAULT,
        preferred_element_type=jnp.float32,
    )
    o_ref[...] = acc + b_ref[...]


def kernel(x, adj, w, b):
    n, f = x.shape
    h = w.shape[1]
    xw = pl.pallas_call(
        _xw_kernel,
        out_shape=jax.ShapeDtypeStruct((n, h), jnp.float32),
    )(x, w)

    bm = 200
    b2 = b.reshape(1, h)
    out = pl.pallas_call(
        _spmm_kernel,
        grid=(pl.cdiv(n, bm),),
        in_specs=[
            pl.BlockSpec((bm, n), lambda i: (i, 0)),
            pl.BlockSpec((n, h), lambda i: (0, 0)),
            pl.BlockSpec((1, h), lambda i: (0, 0)),
        ],
        out_specs=pl.BlockSpec((bm, h), lambda i: (i, 0)),
        out_shape=jax.ShapeDtypeStruct((n, h), jnp.float32),
        compiler_params=pltpu.CompilerParams(
            dimension_semantics=("parallel",),
        ),
    )(adj, xw, b2)
    return out
